# baseline (speedup 1.0000x reference)
import functools

import jax
import jax.numpy as jnp
from jax.experimental import pallas as pl
from jax.experimental.pallas import tpu as pltpu

N_NODES = 100000
D_FEAT = 128
D_HID = 512
D_ATT = 256
N_CLASSES = 4
N_SEG = 16

TILE = 2000
NT = N_NODES // TILE

_DN0 = (((0,), (0,)), ((), ()))


def _fused_kernel(x_ref, bcol_ref, whT_ref, bh_ref, waT_ref, ba_ref,
                  wbT_ref, bb_ref, wcT_ref, bc_ref, wclsT_ref, bcls_ref,
                  out_ref, m_acc, den_acc, pooledT_acc):
    i = pl.program_id(0)

    @pl.when(i == 0)
    def _init():
        m_acc[...] = jnp.full((1, N_SEG), -jnp.inf, dtype=jnp.float32)
        den_acc[...] = jnp.zeros((1, N_SEG), dtype=jnp.float32)
        pooledT_acc[...] = jnp.zeros((D_HID, N_SEG), dtype=jnp.float32)

    x_t = x_ref[...]
    h = jnp.maximum(
        jax.lax.dot_general(x_t, whT_ref[...], (((1,), (0,)), ((), ())),
                            preferred_element_type=jnp.float32)
        + bh_ref[...], 0.0)
    h_bf = h.astype(jnp.bfloat16)
    a = jnp.tanh(
        jax.lax.dot_general(h_bf, waT_ref[...], (((1,), (0,)), ((), ())),
                            preferred_element_type=jnp.float32)
        + ba_ref[...])
    g = jax.nn.sigmoid(
        jax.lax.dot_general(h_bf, wbT_ref[...], (((1,), (0,)), ((), ())),
                            preferred_element_type=jnp.float32)
        + bb_ref[...])
    ag = (a * g).astype(jnp.bfloat16)
    gate = (jax.lax.dot_general(ag, wcT_ref[...], (((1,), (0,)), ((), ())),
                                preferred_element_type=jnp.float32)
            + bc_ref[...])

    bcol = bcol_ref[...]
    seg = jax.lax.broadcasted_iota(jnp.int32, (TILE, N_SEG), 1).astype(
        jnp.float32)
    onehot = bcol == seg

    gate_m = jnp.where(onehot, gate, -jnp.inf)
    m_tile = jnp.max(gate_m, axis=0, keepdims=True)

    m_old = m_acc[...]
    m_new = jnp.maximum(m_old, m_tile)
    scale = jnp.where(m_old > -jnp.inf, jnp.exp(m_old - m_new), 0.0)

    e_w = jnp.where(onehot, jnp.exp(gate - m_new), 0.0)
    den_tile = jnp.sum(e_w, axis=0, keepdims=True)
    contribT = jax.lax.dot_general(h_bf, e_w.astype(jnp.bfloat16), _DN0,
                                   preferred_element_type=jnp.float32)

    m_acc[...] = m_new
    den_acc[...] = den_acc[...] * scale + den_tile
    pooledT_acc[...] = pooledT_acc[...] * scale + contribT

    @pl.when(i == NT - 1)
    def _finish():
        den = den_acc[...]
        recip = jnp.where(den > 0, 1.0 / den, 0.0)
        pT = pooledT_acc[...] * recip
        out_ref[...] = (
            jax.lax.dot_general(pT, wclsT_ref[...], _DN0,
                                preferred_element_type=jnp.float32)
            + bcls_ref[...])


@functools.partial(jax.jit, static_argnames=())
def kernel(x, edge_index, batch, W_head, b_head, Wa, ba, Wb, bb, Wc, bc,
           W_cls, b_cls):
    del edge_index
    bcol = batch.astype(jnp.float32)[:, None]
    x = x.astype(jnp.bfloat16)
    whT = W_head.T.astype(jnp.bfloat16)
    waT = Wa.T.astype(jnp.bfloat16)
    wbT = Wb.T.astype(jnp.bfloat16)
    wcT = Wc.T.astype(jnp.bfloat16)
    wclsT = W_cls.T
    bh = b_head[None, :]
    ba2 = ba[None, :]
    bb2 = bb[None, :]
    bc2 = bc[None, :]
    bcls2 = b_cls[None, :]

    full = lambda shape: pl.BlockSpec(shape, lambda i: (0, 0))
    out = pl.pallas_call(
        _fused_kernel,
        grid=(NT,),
        in_specs=[
            pl.BlockSpec((TILE, D_FEAT), lambda i: (i, 0)),
            pl.BlockSpec((TILE, 1), lambda i: (i, 0)),
            full((D_FEAT, D_HID)),
            full((1, D_HID)),
            full((D_HID, D_ATT)),
            full((1, D_ATT)),
            full((D_HID, D_ATT)),
            full((1, D_ATT)),
            full((D_ATT, 1)),
            full((1, 1)),
            full((D_HID, N_CLASSES)),
            full((1, N_CLASSES)),
        ],
        out_specs=pl.BlockSpec((N_SEG, N_CLASSES), lambda i: (0, 0)),
        out_shape=jax.ShapeDtypeStruct((N_SEG, N_CLASSES), jnp.float32),
        scratch_shapes=[
            pltpu.VMEM((1, N_SEG), jnp.float32),
            pltpu.VMEM((1, N_SEG), jnp.float32),
            pltpu.VMEM((D_HID, N_SEG), jnp.float32),
        ],
    )(x, bcol, whT, bh, waT, ba2, wbT, bb2, wcT, bc2, wclsT, bcls2)
    return out


# speedup vs baseline: 1.1568x; 1.1568x over previous
import functools

import jax
import jax.numpy as jnp
from jax.experimental import pallas as pl
from jax.experimental.pallas import tpu as pltpu

N_NODES = 100000
D_FEAT = 128
D_HID = 512
D_ATT = 256
N_CLASSES = 4
N_SEG = 16

TILE = 2000
NT = N_NODES // TILE

_DN0 = (((0,), (0,)), ((), ()))


def _fused_kernel(x_ref, bcol_ref, whT_ref, bh_ref, waT_ref, ba_ref,
                  wbT_ref, bb_ref, wcT_ref, bc_ref, wclsT_ref, bcls_ref,
                  out_ref, m_acc, den_acc, pooled_acc):
    i = pl.program_id(0)

    @pl.when(i == 0)
    def _init():
        m_acc[...] = jnp.full((1, N_SEG), -jnp.inf, dtype=jnp.float32)
        den_acc[...] = jnp.zeros((1, N_SEG), dtype=jnp.float32)
        pooled_acc[...] = jnp.zeros((N_SEG, D_HID), dtype=jnp.float32)

    x_t = x_ref[...]
    h = jnp.maximum(
        jax.lax.dot_general(x_t, whT_ref[...], (((1,), (0,)), ((), ())),
                            preferred_element_type=jnp.float32)
        + bh_ref[...], 0.0)
    a = jnp.tanh(
        jax.lax.dot_general(h, waT_ref[...], (((1,), (0,)), ((), ())),
                            preferred_element_type=jnp.float32)
        + ba_ref[...])
    g = jax.nn.sigmoid(
        jax.lax.dot_general(h, wbT_ref[...], (((1,), (0,)), ((), ())),
                            preferred_element_type=jnp.float32)
        + bb_ref[...])
    gate = (jax.lax.dot_general(a * g, wcT_ref[...], (((1,), (0,)), ((), ())),
                                preferred_element_type=jnp.float32)
            + bc_ref[...])

    bcol = bcol_ref[...]
    seg = jax.lax.broadcasted_iota(jnp.int32, (TILE, N_SEG), 1).astype(
        jnp.float32)
    onehot = bcol == seg

    gate_m = jnp.where(onehot, gate, -jnp.inf)
    m_tile = jnp.max(gate_m, axis=0, keepdims=True)

    m_old = m_acc[...]
    m_new = jnp.maximum(m_old, m_tile)
    scale = jnp.where(m_old > -jnp.inf, jnp.exp(m_old - m_new), 0.0)

    e_w = jnp.where(onehot, jnp.exp(gate - m_new), 0.0)
    den_tile = jnp.sum(e_w, axis=0, keepdims=True)
    contrib = jax.lax.dot_general(e_w, h, _DN0,
                                  preferred_element_type=jnp.float32)

    eye = (jax.lax.broadcasted_iota(jnp.int32, (N_SEG, N_SEG), 0)
           == jax.lax.broadcasted_iota(jnp.int32, (N_SEG, N_SEG), 1)
           ).astype(jnp.float32)
    scale_col = jax.lax.dot_general(eye, scale, (((1,), (1,)), ((), ())),
                                    preferred_element_type=jnp.float32)

    m_acc[...] = m_new
    den_acc[...] = den_acc[...] * scale + den_tile
    pooled_acc[...] = pooled_acc[...] * scale_col + contrib

    @pl.when(i == NT - 1)
    def _finish():
        den = den_acc[...]
        recip = jnp.where(den > 0, 1.0 / den, 0.0)
        recip_col = jax.lax.dot_general(
            eye, recip, (((1,), (1,)), ((), ())),
            preferred_element_type=jnp.float32)
        pooled = pooled_acc[...] * recip_col
        out_ref[...] = (
            jax.lax.dot_general(pooled, wclsT_ref[...],
                                (((1,), (0,)), ((), ())),
                                preferred_element_type=jnp.float32)
            + bcls_ref[...])


@functools.partial(jax.jit, static_argnames=())
def kernel(x, edge_index, batch, W_head, b_head, Wa, ba, Wb, bb, Wc, bc,
           W_cls, b_cls):
    del edge_index
    bcol = batch.astype(jnp.float32)[:, None]
    whT = W_head.T
    waT = Wa.T
    wbT = Wb.T
    wcT = Wc.T
    wclsT = W_cls.T
    bh = b_head[None, :]
    ba2 = ba[None, :]
    bb2 = bb[None, :]
    bc2 = bc[None, :]
    bcls2 = b_cls[None, :]

    full = lambda shape: pl.BlockSpec(shape, lambda i: (0, 0))
    out = pl.pallas_call(
        _fused_kernel,
        grid=(NT,),
        in_specs=[
            pl.BlockSpec((TILE, D_FEAT), lambda i: (i, 0)),
            pl.BlockSpec((TILE, 1), lambda i: (i, 0)),
            full((D_FEAT, D_HID)),
            full((1, D_HID)),
            full((D_HID, D_ATT)),
            full((1, D_ATT)),
            full((D_HID, D_ATT)),
            full((1, D_ATT)),
            full((D_ATT, 1)),
            full((1, 1)),
            full((D_HID, N_CLASSES)),
            full((1, N_CLASSES)),
        ],
        out_specs=pl.BlockSpec((N_SEG, N_CLASSES), lambda i: (0, 0)),
        out_shape=jax.ShapeDtypeStruct((N_SEG, N_CLASSES), jnp.float32),
        scratch_shapes=[
            pltpu.VMEM((1, N_SEG), jnp.float32),
            pltpu.VMEM((1, N_SEG), jnp.float32),
            pltpu.VMEM((N_SEG, D_HID), jnp.float32),
        ],
    )(x, bcol, whT, bh, waT, ba2, wbT, bb2, wcT, bc2, wclsT, bcls2)
    return out


# speedup vs baseline: 1.1670x; 1.0088x over previous
import functools

import jax
import jax.numpy as jnp
from jax.experimental import pallas as pl
from jax.experimental.pallas import tpu as pltpu

N_NODES = 100000
D_FEAT = 128
D_HID = 512
D_ATT = 256
N_CLASSES = 4
N_SEG = 16

TILE = 4000
NT = N_NODES // TILE

_DN0 = (((0,), (0,)), ((), ()))


def _fused_kernel(x_ref, bcol_ref, whT_ref, bh_ref, waT_ref, ba_ref,
                  wbT_ref, bb_ref, wcT_ref, bc_ref, wclsT_ref, bcls_ref,
                  out_ref, m_acc, den_acc, pooled_acc):
    i = pl.program_id(0)

    @pl.when(i == 0)
    def _init():
        m_acc[...] = jnp.full((1, N_SEG), -jnp.inf, dtype=jnp.float32)
        den_acc[...] = jnp.zeros((1, N_SEG), dtype=jnp.float32)
        pooled_acc[...] = jnp.zeros((N_SEG, D_HID), dtype=jnp.float32)

    x_t = x_ref[...]
    h = jnp.maximum(
        jax.lax.dot_general(x_t, whT_ref[...], (((1,), (0,)), ((), ())),
                            preferred_element_type=jnp.float32)
        + bh_ref[...], 0.0)
    a = jnp.tanh(
        jax.lax.dot_general(h, waT_ref[...], (((1,), (0,)), ((), ())),
                            preferred_element_type=jnp.float32)
        + ba_ref[...])
    g = jax.nn.sigmoid(
        jax.lax.dot_general(h, wbT_ref[...], (((1,), (0,)), ((), ())),
                            preferred_element_type=jnp.float32)
        + bb_ref[...])
    gate = (jax.lax.dot_general(a * g, wcT_ref[...], (((1,), (0,)), ((), ())),
                                preferred_element_type=jnp.float32)
            + bc_ref[...])

    bcol = bcol_ref[...]
    seg = jax.lax.broadcasted_iota(jnp.int32, (TILE, N_SEG), 1).astype(
        jnp.float32)
    onehot = bcol == seg

    gate_m = jnp.where(onehot, gate, -jnp.inf)
    m_tile = jnp.max(gate_m, axis=0, keepdims=True)

    m_old = m_acc[...]
    m_new = jnp.maximum(m_old, m_tile)
    scale = jnp.where(m_old > -jnp.inf, jnp.exp(m_old - m_new), 0.0)

    e_w = jnp.where(onehot, jnp.exp(gate - m_new), 0.0)
    den_tile = jnp.sum(e_w, axis=0, keepdims=True)
    contrib = jax.lax.dot_general(e_w, h, _DN0,
                                  preferred_element_type=jnp.float32)

    eye = (jax.lax.broadcasted_iota(jnp.int32, (N_SEG, N_SEG), 0)
           == jax.lax.broadcasted_iota(jnp.int32, (N_SEG, N_SEG), 1)
           ).astype(jnp.float32)
    scale_col = jax.lax.dot_general(eye, scale, (((1,), (1,)), ((), ())),
                                    preferred_element_type=jnp.float32)

    m_acc[...] = m_new
    den_acc[...] = den_acc[...] * scale + den_tile
    pooled_acc[...] = pooled_acc[...] * scale_col + contrib

    @pl.when(i == NT - 1)
    def _finish():
        den = den_acc[...]
        recip = jnp.where(den > 0, 1.0 / den, 0.0)
        recip_col = jax.lax.dot_general(
            eye, recip, (((1,), (1,)), ((), ())),
            preferred_element_type=jnp.float32)
        pooled = pooled_acc[...] * recip_col
        out_ref[...] = (
            jax.lax.dot_general(pooled, wclsT_ref[...],
                                (((1,), (0,)), ((), ())),
                                preferred_element_type=jnp.float32)
            + bcls_ref[...])


@functools.partial(jax.jit, static_argnames=())
def kernel(x, edge_index, batch, W_head, b_head, Wa, ba, Wb, bb, Wc, bc,
           W_cls, b_cls):
    del edge_index
    bcol = batch.astype(jnp.float32)[:, None]
    whT = W_head.T
    waT = Wa.T
    wbT = Wb.T
    wcT = Wc.T
    wclsT = W_cls.T
    bh = b_head[None, :]
    ba2 = ba[None, :]
    bb2 = bb[None, :]
    bc2 = bc[None, :]
    bcls2 = b_cls[None, :]

    full = lambda shape: pl.BlockSpec(shape, lambda i: (0, 0))
    out = pl.pallas_call(
        _fused_kernel,
        grid=(NT,),
        in_specs=[
            pl.BlockSpec((TILE, D_FEAT), lambda i: (i, 0)),
            pl.BlockSpec((TILE, 1), lambda i: (i, 0)),
            full((D_FEAT, D_HID)),
            full((1, D_HID)),
            full((D_HID, D_ATT)),
            full((1, D_ATT)),
            full((D_HID, D_ATT)),
            full((1, D_ATT)),
            full((D_ATT, 1)),
            full((1, 1)),
            full((D_HID, N_CLASSES)),
            full((1, N_CLASSES)),
        ],
        out_specs=pl.BlockSpec((N_SEG, N_CLASSES), lambda i: (0, 0)),
        out_shape=jax.ShapeDtypeStruct((N_SEG, N_CLASSES), jnp.float32),
        scratch_shapes=[
            pltpu.VMEM((1, N_SEG), jnp.float32),
            pltpu.VMEM((1, N_SEG), jnp.float32),
            pltpu.VMEM((N_SEG, D_HID), jnp.float32),
        ],
    )(x, bcol, whT, bh, waT, ba2, wbT, bb2, wcT, bc2, wclsT, bcls2)
    return out


# speedup vs baseline: 1.7821x; 1.5271x over previous
import functools

import jax
import jax.numpy as jnp
from jax.experimental import pallas as pl
from jax.experimental.pallas import tpu as pltpu

N_NODES = 100000
D_FEAT = 128
D_HID = 512
D_ATT = 256
N_CLASSES = 4
N_SEG = 16

TILE = 4000
NT = N_NODES // TILE


def _fused_kernel(x_ref, brow_ref, whT_ref, bh_ref, wabT_ref, bab_ref,
                  wc_ref, bc_ref, wclsT_ref, bcls_ref,
                  out_ref, m_acc, den_acc, pooled_acc):
    i = pl.program_id(0)

    @pl.when(i == 0)
    def _init():
        m_acc[...] = jnp.full((N_SEG, 1), -jnp.inf, dtype=jnp.float32)
        den_acc[...] = jnp.zeros((N_SEG, 1), dtype=jnp.float32)
        pooled_acc[...] = jnp.zeros((N_SEG, D_HID), dtype=jnp.float32)

    x_t = x_ref[...]
    h = jnp.maximum(
        jax.lax.dot_general(x_t, whT_ref[...], (((1,), (0,)), ((), ())),
                            preferred_element_type=jnp.float32)
        + bh_ref[...], 0.0)
    ab = (jax.lax.dot_general(h, wabT_ref[...], (((1,), (0,)), ((), ())),
                              preferred_element_type=jnp.float32)
          + bab_ref[...])
    ag = jnp.tanh(ab[:, :D_ATT]) * jax.nn.sigmoid(ab[:, D_ATT:])
    gate = (jax.lax.dot_general(wc_ref[...], ag, (((1,), (1,)), ((), ())),
                                preferred_element_type=jnp.float32)
            + bc_ref[...])

    brow = brow_ref[0]
    seg = jax.lax.broadcasted_iota(jnp.int32, (N_SEG, TILE), 0).astype(
        jnp.float32)
    onehot = brow == seg

    gate_m = jnp.where(onehot, gate, -jnp.inf)
    m_tile = jnp.max(gate_m, axis=1, keepdims=True)

    m_old = m_acc[...]
    m_new = jnp.maximum(m_old, m_tile)
    scale = jnp.where(m_old > -jnp.inf, jnp.exp(m_old - m_new), 0.0)

    e_w = jnp.where(onehot, jnp.exp(gate - m_new), 0.0)
    den_tile = jnp.sum(e_w, axis=1, keepdims=True)
    contrib = jax.lax.dot_general(e_w, h, (((1,), (0,)), ((), ())),
                                  preferred_element_type=jnp.float32)

    m_acc[...] = m_new
    den_acc[...] = den_acc[...] * scale + den_tile
    pooled_acc[...] = pooled_acc[...] * scale + contrib

    @pl.when(i == NT - 1)
    def _finish():
        den = den_acc[...]
        recip = jnp.where(den > 0, 1.0 / den, 0.0)
        pooled = pooled_acc[...] * recip
        out_ref[...] = (
            jax.lax.dot_general(pooled, wclsT_ref[...],
                                (((1,), (0,)), ((), ())),
                                preferred_element_type=jnp.float32)
            + bcls_ref[...])


@functools.partial(jax.jit, static_argnames=())
def kernel(x, edge_index, batch, W_head, b_head, Wa, ba, Wb, bb, Wc, bc,
           W_cls, b_cls):
    del edge_index
    brow = batch.astype(jnp.float32).reshape(NT, 1, TILE)
    whT = W_head.T
    wabT = jnp.concatenate([Wa.T, Wb.T], axis=1)
    bab = jnp.concatenate([ba, bb])[None, :]
    wclsT = W_cls.T
    bh = b_head[None, :]
    bc2 = bc[None, :]
    bcls2 = b_cls[None, :]

    out = pl.pallas_call(
        _fused_kernel,
        grid=(NT,),
        in_specs=[
            pl.BlockSpec((TILE, D_FEAT), lambda i: (i, 0)),
            pl.BlockSpec((1, 1, TILE), lambda i: (i, 0, 0)),
            pl.BlockSpec((D_FEAT, D_HID), lambda i: (0, 0)),
            pl.BlockSpec((1, D_HID), lambda i: (0, 0)),
            pl.BlockSpec((D_HID, 2 * D_ATT), lambda i: (0, 0)),
            pl.BlockSpec((1, 2 * D_ATT), lambda i: (0, 0)),
            pl.BlockSpec((1, D_ATT), lambda i: (0, 0)),
            pl.BlockSpec((1, 1), lambda i: (0, 0)),
            pl.BlockSpec((D_HID, N_CLASSES), lambda i: (0, 0)),
            pl.BlockSpec((1, N_CLASSES), lambda i: (0, 0)),
        ],
        out_specs=pl.BlockSpec((N_SEG, N_CLASSES), lambda i: (0, 0)),
        out_shape=jax.ShapeDtypeStruct((N_SEG, N_CLASSES), jnp.float32),
        scratch_shapes=[
            pltpu.VMEM((N_SEG, 1), jnp.float32),
            pltpu.VMEM((N_SEG, 1), jnp.float32),
            pltpu.VMEM((N_SEG, D_HID), jnp.float32),
        ],
    )(x, brow, whT, bh, wabT, bab, Wc, bc2, wclsT, bcls2)
    return out


# speedup vs baseline: 1.8563x; 1.0416x over previous
import functools

import jax
import jax.numpy as jnp
from jax.experimental import pallas as pl
from jax.experimental.pallas import tpu as pltpu

N_NODES = 100000
D_FEAT = 128
D_HID = 512
D_ATT = 256
N_CLASSES = 4
N_SEG = 16

TILE = 4000
NT = N_NODES // TILE


def _fused_kernel(x_ref, brow_ref, whT_ref, bh_ref, wabT_ref, bab_ref,
                  wc_ref, bcs_ref, wclsT_ref, bcls_ref,
                  out_ref, den_acc, pooled_acc):
    i = pl.program_id(0)

    @pl.when(i == 0)
    def _init():
        den_acc[...] = jnp.zeros((N_SEG, 1), dtype=jnp.float32)
        pooled_acc[...] = jnp.zeros((N_SEG, D_HID), dtype=jnp.float32)

    x_t = x_ref[...]
    h = jnp.maximum(
        jax.lax.dot_general(x_t, whT_ref[...], (((1,), (0,)), ((), ())),
                            preferred_element_type=jnp.float32)
        + bh_ref[...], 0.0)
    ab = (jax.lax.dot_general(h, wabT_ref[...], (((1,), (0,)), ((), ())),
                              preferred_element_type=jnp.float32)
          + bab_ref[...])
    ag = jnp.tanh(ab[:, :D_ATT]) * jax.nn.sigmoid(ab[:, D_ATT:])
    gate = (jax.lax.dot_general(wc_ref[...], ag, (((1,), (1,)), ((), ())),
                                preferred_element_type=jnp.float32)
            + bcs_ref[...])
    e_row = jnp.exp(gate)

    brow = brow_ref[0]
    seg = jax.lax.broadcasted_iota(jnp.int32, (N_SEG, TILE), 0).astype(
        jnp.float32)
    e_w = jnp.where(brow == seg, e_row, 0.0)

    den_tile = jnp.sum(e_w, axis=1, keepdims=True)
    contrib = jax.lax.dot_general(e_w, h, (((1,), (0,)), ((), ())),
                                  preferred_element_type=jnp.float32)

    den_acc[...] = den_acc[...] + den_tile
    pooled_acc[...] = pooled_acc[...] + contrib

    @pl.when(i == NT - 1)
    def _finish():
        den = den_acc[...]
        recip = jnp.where(den > 0, 1.0 / den, 0.0)
        pooled = pooled_acc[...] * recip
        out_ref[...] = (
            jax.lax.dot_general(pooled, wclsT_ref[...],
                                (((1,), (0,)), ((), ())),
                                preferred_element_type=jnp.float32)
            + bcls_ref[...])


@functools.partial(jax.jit, static_argnames=())
def kernel(x, edge_index, batch, W_head, b_head, Wa, ba, Wb, bb, Wc, bc,
           W_cls, b_cls):
    del edge_index
    brow = batch.astype(jnp.float32).reshape(NT, 1, TILE)
    whT = W_head.T
    wabT = jnp.concatenate([Wa.T, Wb.T], axis=1)
    bab = jnp.concatenate([ba, bb])[None, :]
    wclsT = W_cls.T
    bh = b_head[None, :]
    shift = jnp.sum(jnp.abs(Wc)) + jnp.abs(bc[0])
    bcs = (bc - shift)[None, :]
    bcls2 = b_cls[None, :]

    out = pl.pallas_call(
        _fused_kernel,
        grid=(NT,),
        in_specs=[
            pl.BlockSpec((TILE, D_FEAT), lambda i: (i, 0)),
            pl.BlockSpec((1, 1, TILE), lambda i: (i, 0, 0)),
            pl.BlockSpec((D_FEAT, D_HID), lambda i: (0, 0)),
            pl.BlockSpec((1, D_HID), lambda i: (0, 0)),
            pl.BlockSpec((D_HID, 2 * D_ATT), lambda i: (0, 0)),
            pl.BlockSpec((1, 2 * D_ATT), lambda i: (0, 0)),
            pl.BlockSpec((1, D_ATT), lambda i: (0, 0)),
            pl.BlockSpec((1, 1), lambda i: (0, 0)),
            pl.BlockSpec((D_HID, N_CLASSES), lambda i: (0, 0)),
            pl.BlockSpec((1, N_CLASSES), lambda i: (0, 0)),
        ],
        out_specs=pl.BlockSpec((N_SEG, N_CLASSES), lambda i: (0, 0)),
        out_shape=jax.ShapeDtypeStruct((N_SEG, N_CLASSES), jnp.float32),
        scratch_shapes=[
            pltpu.VMEM((N_SEG, 1), jnp.float32),
            pltpu.VMEM((N_SEG, D_HID), jnp.float32),
        ],
    )(x, brow, whT, bh, wabT, bab, Wc, bcs, wclsT, bcls2)
    return out
